# baseline (speedup 1.0000x reference)
import functools

import jax
import jax.numpy as jnp
from jax import lax
from jax.experimental import pallas as pl
from jax.experimental.pallas import tpu as pltpu
from jax.experimental.pallas import tpu_sc as plsc

B, S, D_MODEL, D_FF, E = 1, 2048, 1024, 4096, 8
TB = 128
NBP = S // TB + (E - 1)
NROWS = NBP * TB
FFT = 1024
NFF = D_FF // FFT
NMETA = 32


def _plan_body(x_ref, wr_ref, br_ref, dest_ref, meta_ref):
    logits = jnp.dot(x_ref[...], wr_ref[...],
                     preferred_element_type=jnp.float32) + br_ref[...]
    m = jnp.max(logits, axis=1, keepdims=True)
    e_iota = lax.broadcasted_iota(jnp.int32, (S, E), 1)
    idx = jnp.min(jnp.where(logits >= m, e_iota, E), axis=1, keepdims=True)
    oh = (e_iota == idx).astype(jnp.float32)

    counts = jnp.sum(oh, axis=0, keepdims=True).astype(jnp.int32)
    cpad = ((counts + TB - 1) // TB) * TB
    er = lax.broadcasted_iota(jnp.int32, (E, E), 0)
    ec = lax.broadcasted_iota(jnp.int32, (E, E), 1)
    offs = jnp.dot(cpad.astype(jnp.float32), (er < ec).astype(jnp.float32),
                   preferred_element_type=jnp.float32).astype(jnp.int32)

    C = 512
    G = S // C
    oh3 = oh.reshape(G, C, E)
    csum = jnp.sum(oh3, axis=1)
    gr = lax.broadcasted_iota(jnp.int32, (G, G), 0)
    gc = lax.broadcasted_iota(jnp.int32, (G, G), 1)
    base = jnp.dot((gr > gc).astype(jnp.float32), csum,
                   preferred_element_type=jnp.float32)
    rr = lax.broadcasted_iota(jnp.int32, (C, C), 0)
    rc = lax.broadcasted_iota(jnp.int32, (C, C), 1)
    ltri = (rr > rc).astype(jnp.float32)
    ranks = [jnp.dot(ltri, oh3[g], preferred_element_type=jnp.float32)
             + base[g][None, :] for g in range(G)]
    rank_full = jnp.concatenate(ranks, axis=0)
    rank = jnp.sum(rank_full * oh, axis=1, keepdims=True)

    dest_off = jnp.sum(offs.astype(jnp.float32) * oh, axis=1, keepdims=True)
    dest_ref[...] = (dest_off + rank).astype(jnp.int32)

    nblk = jnp.sum(cpad, axis=1, keepdims=True) // TB
    blk_start = offs // TB
    b_iota = lax.broadcasted_iota(jnp.int32, (NMETA, 1), 0)
    b_cl = jnp.minimum(b_iota, nblk - 1)
    blk_e = jnp.sum((b_cl >= blk_start).astype(jnp.int32),
                    axis=1, keepdims=True) - 1
    valid = (b_iota < nblk).astype(jnp.int32)
    pad = jnp.zeros((NMETA, E - 3), jnp.int32)
    meta_ref[...] = jnp.concatenate([blk_e, b_cl, valid, pad], axis=1)


def _plan(xf, Wr, br2):
    return pl.pallas_call(
        _plan_body,
        out_shape=(jax.ShapeDtypeStruct((S, 1), jnp.int32),
                   jax.ShapeDtypeStruct((NMETA, E), jnp.int32)),
    )(xf, Wr, br2)


def _sc_mesh():
    return plsc.VectorSubcoreMesh(core_axis_name="c", subcore_axis_name="s")


def _dispatch(xf, dest):
    info = plsc.get_sparse_core_info()
    nw = info.num_cores * info.num_subcores
    rows_w = S // nw

    @functools.partial(
        pl.kernel, mesh=_sc_mesh(),
        out_type=jax.ShapeDtypeStruct((NROWS, D_MODEL), jnp.float32),
        scratch_types=[pltpu.VMEM((rows_w,), jnp.int32),
                       pltpu.VMEM((rows_w, D_MODEL), jnp.float32),
                       pltpu.SemaphoreType.DMA],
    )
    def k(x_hbm, d_hbm, out_hbm, idx_v, rows_v, sem):
        wid = lax.axis_index("s") * info.num_cores + lax.axis_index("c")
        base = wid * rows_w
        pltpu.sync_copy(d_hbm.at[pl.ds(base, rows_w)], idx_v)
        pltpu.sync_copy(x_hbm.at[pl.ds(base, rows_w)], rows_v)
        pltpu.async_copy(rows_v, out_hbm.at[idx_v], sem).wait()

    return k(xf, dest)


def _combine(sorted_out, dest):
    info = plsc.get_sparse_core_info()
    nw = info.num_cores * info.num_subcores
    rows_w = S // nw

    @functools.partial(
        pl.kernel, mesh=_sc_mesh(),
        out_type=jax.ShapeDtypeStruct((S, D_MODEL), jnp.float32),
        scratch_types=[pltpu.VMEM((rows_w,), jnp.int32),
                       pltpu.VMEM((rows_w, D_MODEL), jnp.float32),
                       pltpu.SemaphoreType.DMA],
    )
    def k(s_hbm, d_hbm, out_hbm, idx_v, rows_v, sem):
        wid = lax.axis_index("s") * info.num_cores + lax.axis_index("c")
        base = wid * rows_w
        pltpu.sync_copy(d_hbm.at[pl.ds(base, rows_w)], idx_v)
        pltpu.async_copy(s_hbm.at[idx_v], rows_v, sem).wait()
        pltpu.sync_copy(rows_v, out_hbm.at[pl.ds(base, rows_w)])

    return k(sorted_out, dest)


def _ffn_body(e_ref, xb_ref, v_ref, xs_ref, w1_ref, b1_ref, w2_ref, b2_ref,
              out_ref, acc_ref):
    j = pl.program_id(0)
    b = pl.program_id(1)

    @pl.when(v_ref[b] > 0)
    def _():
        h = jnp.maximum(
            jnp.dot(xs_ref[...], w1_ref[0],
                    preferred_element_type=jnp.float32,
                    precision=lax.Precision.DEFAULT) + b1_ref[0, 0], 0.0)
        p = jnp.dot(h, w2_ref[0], preferred_element_type=jnp.float32,
                    precision=lax.Precision.DEFAULT)
        row = xb_ref[b] * TB

        @pl.when(j == 0)
        def _():
            acc_ref[pl.ds(row, TB), :] = p

        @pl.when(j > 0)
        def _():
            acc_ref[pl.ds(row, TB), :] += p

        @pl.when(j == NFF - 1)
        def _():
            out_ref[...] = acc_ref[pl.ds(row, TB), :] + b2_ref[0]


def _ffn(xs, W1, b1, W2, b2, blk_e, blk_xb, blk_v):
    grid_spec = pltpu.PrefetchScalarGridSpec(
        num_scalar_prefetch=3,
        grid=(NFF, NBP),
        in_specs=[
            pl.BlockSpec((TB, D_MODEL),
                         lambda j, b, e, xb, v: (xb[b], 0)),
            pl.BlockSpec((1, D_MODEL, FFT),
                         lambda j, b, e, xb, v: (e[b], 0, j)),
            pl.BlockSpec((1, 1, 1, FFT),
                         lambda j, b, e, xb, v: (e[b], j, 0, 0)),
            pl.BlockSpec((1, FFT, D_MODEL),
                         lambda j, b, e, xb, v: (e[b], j, 0)),
            pl.BlockSpec((1, 1, D_MODEL),
                         lambda j, b, e, xb, v: (e[b], 0, 0)),
        ],
        out_specs=pl.BlockSpec(
            (TB, D_MODEL),
            lambda j, b, e, xb, v: (jnp.where(j == NFF - 1, xb[b], 0), 0)),
        scratch_shapes=[pltpu.VMEM((NROWS, D_MODEL), jnp.float32)],
    )
    return pl.pallas_call(
        _ffn_body,
        grid_spec=grid_spec,
        out_shape=jax.ShapeDtypeStruct((NROWS, D_MODEL), jnp.float32),
        compiler_params=pltpu.CompilerParams(
            dimension_semantics=("arbitrary", "arbitrary")),
    )(blk_e, blk_xb, blk_v, xs, W1,
      b1.reshape(E, NFF, 1, FFT), W2, b2.reshape(E, 1, D_MODEL))


def kernel(x, W1, b1, W2, b2, Wr, br):
    xf = x.reshape(S, D_MODEL)
    dest2d, meta = _plan(xf, Wr, br.reshape(1, E))
    dest = dest2d.reshape(S)
    xs = _dispatch(xf, dest)
    outs = _ffn(xs, W1, b1, W2, b2, meta[:NBP, 0], meta[:NBP, 1],
                meta[:NBP, 2])
    out = _combine(outs, dest)
    return out.reshape(B, S, D_MODEL)


# speedup vs baseline: 1.3319x; 1.3319x over previous
import functools

import jax
import jax.numpy as jnp
from jax import lax
from jax.experimental import pallas as pl
from jax.experimental.pallas import tpu as pltpu
from jax.experimental.pallas import tpu_sc as plsc

B, S, D_MODEL, D_FF, E = 1, 2048, 1024, 4096, 8
TB = 128
NBP = S // TB + (E - 1)
NROWS = NBP * TB
FFT = 1024
NFF = D_FF // FFT
NMETA = 32


def _plan_body(x_ref, wr_ref, br_ref, dest_ref, meta_ref, ri_ref):
    logits = jnp.dot(x_ref[...], wr_ref[...],
                     preferred_element_type=jnp.float32) + br_ref[...]
    m = jnp.max(logits, axis=1, keepdims=True)
    e_iota = lax.broadcasted_iota(jnp.int32, (S, E), 1)
    idx = jnp.min(jnp.where(logits >= m, e_iota, E), axis=1, keepdims=True)
    oh = (e_iota == idx).astype(jnp.float32)

    counts = jnp.sum(oh, axis=0, keepdims=True).astype(jnp.int32)
    cpad = ((counts + TB - 1) // TB) * TB
    er = lax.broadcasted_iota(jnp.int32, (E, E), 0)
    ec = lax.broadcasted_iota(jnp.int32, (E, E), 1)
    offs = jnp.dot(cpad.astype(jnp.float32), (er < ec).astype(jnp.float32),
                   preferred_element_type=jnp.float32).astype(jnp.int32)

    C = 512
    G = S // C
    oh3 = oh.reshape(G, C, E)
    csum = jnp.sum(oh3, axis=1)
    gr = lax.broadcasted_iota(jnp.int32, (G, G), 0)
    gc = lax.broadcasted_iota(jnp.int32, (G, G), 1)
    base = jnp.dot((gr > gc).astype(jnp.float32), csum,
                   preferred_element_type=jnp.float32)
    rr = lax.broadcasted_iota(jnp.int32, (C, C), 0)
    rc = lax.broadcasted_iota(jnp.int32, (C, C), 1)
    ltri = (rr > rc).astype(jnp.float32)
    ranks = [jnp.dot(ltri, oh3[g], preferred_element_type=jnp.float32)
             + base[g][None, :] for g in range(G)]
    rank_full = jnp.concatenate(ranks, axis=0)
    rank = jnp.sum(rank_full * oh, axis=1, keepdims=True)

    dest_off = jnp.sum(offs.astype(jnp.float32) * oh, axis=1, keepdims=True)
    dest_ref[...] = (dest_off + rank).astype(jnp.int32)

    nblk = jnp.sum(cpad, axis=1, keepdims=True) // TB
    blk_start = offs // TB
    b_iota = lax.broadcasted_iota(jnp.int32, (NMETA, 1), 0)
    b_cl = jnp.minimum(b_iota, nblk - 1)
    blk_e = jnp.sum((b_cl >= blk_start).astype(jnp.int32),
                    axis=1, keepdims=True) - 1
    valid = (b_iota < nblk).astype(jnp.int32)

    present = (counts > 0).astype(jnp.float32)
    pprefix = jnp.dot(present, (er < ec).astype(jnp.float32),
                      preferred_element_type=jnp.float32)
    oh_blk = (blk_e == lax.broadcasted_iota(jnp.int32, (NMETA, E), 1)
              ).astype(jnp.float32)
    run_of = jnp.sum(oh_blk * pprefix, axis=1, keepdims=True).astype(jnp.int32)
    bstart_of = jnp.sum(oh_blk * blk_start.astype(jnp.float32),
                        axis=1, keepdims=True).astype(jnp.int32)
    first = ((b_cl == bstart_of) & (valid > 0)).astype(jnp.int32)
    pad = jnp.zeros((NMETA, E - 5), jnp.int32)
    meta_ref[...] = jnp.concatenate(
        [blk_e, b_cl, valid, first, run_of, pad], axis=1)

    pp_i = pprefix.astype(jnp.int32)
    riota = lax.broadcasted_iota(jnp.int32, (E, E), 0)
    eiota8 = lax.broadcasted_iota(jnp.int32, (E, E), 1)
    cmp = ((pp_i == riota) & (counts > 0)).astype(jnp.int32)
    re_col = jnp.sum(cmp * eiota8, axis=1, keepdims=True)
    nruns = jnp.sum(present, axis=1, keepdims=True).astype(jnp.int32)
    ri_ref[...] = jnp.concatenate(
        [re_col, nruns, jnp.zeros((16 - E - 1, 1), jnp.int32)], axis=0)


def _plan(xf, Wr, br2):
    return pl.pallas_call(
        _plan_body,
        out_shape=(jax.ShapeDtypeStruct((S, 1), jnp.int32),
                   jax.ShapeDtypeStruct((NMETA, E), jnp.int32),
                   jax.ShapeDtypeStruct((16, 1), jnp.int32)),
    )(xf, Wr, br2)


def _sc_mesh():
    return plsc.VectorSubcoreMesh(core_axis_name="c", subcore_axis_name="s")


def _dispatch(xf, dest):
    info = plsc.get_sparse_core_info()
    nw = info.num_cores * info.num_subcores
    rows_w = S // nw

    @functools.partial(
        pl.kernel, mesh=_sc_mesh(),
        out_type=jax.ShapeDtypeStruct((NROWS, D_MODEL), jnp.float32),
        scratch_types=[pltpu.VMEM((rows_w,), jnp.int32),
                       pltpu.VMEM((rows_w, D_MODEL), jnp.float32),
                       pltpu.SemaphoreType.DMA],
    )
    def k(x_hbm, d_hbm, out_hbm, idx_v, rows_v, sem):
        wid = lax.axis_index("s") * info.num_cores + lax.axis_index("c")
        base = wid * rows_w
        pltpu.sync_copy(d_hbm.at[pl.ds(base, rows_w)], idx_v)
        pltpu.sync_copy(x_hbm.at[pl.ds(base, rows_w)], rows_v)
        pltpu.async_copy(rows_v, out_hbm.at[idx_v], sem).wait()

    return k(xf, dest)


def _combine(sorted_out, dest):
    info = plsc.get_sparse_core_info()
    nw = info.num_cores * info.num_subcores
    rows_w = S // nw

    @functools.partial(
        pl.kernel, mesh=_sc_mesh(),
        out_type=jax.ShapeDtypeStruct((S, D_MODEL), jnp.float32),
        scratch_types=[pltpu.VMEM((rows_w,), jnp.int32),
                       pltpu.VMEM((rows_w, D_MODEL), jnp.float32),
                       pltpu.SemaphoreType.DMA],
    )
    def k(s_hbm, d_hbm, out_hbm, idx_v, rows_v, sem):
        wid = lax.axis_index("s") * info.num_cores + lax.axis_index("c")
        base = wid * rows_w
        pltpu.sync_copy(d_hbm.at[pl.ds(base, rows_w)], idx_v)
        pltpu.async_copy(s_hbm.at[idx_v], rows_v, sem).wait()
        pltpu.sync_copy(rows_v, out_hbm.at[pl.ds(base, rows_w)])

    return k(sorted_out, dest)


NBUF = 4


def _ffn_body(meta_ref, ri_ref, xs_ref, w1_hbm, b1_ref, w2_hbm, b2_ref,
              out_ref, acc_ref, w1_buf, w2_buf, sems):
    j = pl.program_id(0)
    b = pl.program_id(1)
    nr = ri_ref[E]
    r = meta_ref[b, 4]
    k = j * nr + r
    nseg = NFF * nr

    def seg_copies(kseg, slot):
        jt = kseg // nr
        et = ri_ref[kseg - jt * nr]
        c1 = pltpu.make_async_copy(
            w1_hbm.at[et, :, pl.ds(jt * FFT, FFT)], w1_buf.at[slot],
            sems.at[slot])
        c2 = pltpu.make_async_copy(
            w2_hbm.at[et, pl.ds(jt * FFT, FFT), :], w2_buf.at[slot],
            sems.at[slot])
        return c1, c2

    def issue(kseg):
        c1, c2 = seg_copies(kseg, lax.rem(kseg, NBUF))
        c1.start()
        c2.start()

    is_first = meta_ref[b, 3] > 0

    @pl.when(is_first & (k == 0))
    def _():
        issue(0)
        for d in range(1, NBUF):
            @pl.when(nseg > d)
            def _(d=d):
                issue(d)

    @pl.when(is_first)
    def _():
        c1, c2 = seg_copies(k, lax.rem(k, NBUF))
        c1.wait()
        c2.wait()

        @pl.when((k > 0) & (k + NBUF - 1 < nseg))
        def _():
            issue(k + NBUF - 1)

    @pl.when(meta_ref[b, 2] > 0)
    def _():
        slot = lax.rem(k, NBUF)
        row = meta_ref[b, 1] * TB
        h = jnp.maximum(
            jnp.dot(xs_ref[pl.ds(row, TB), :], w1_buf[slot],
                    preferred_element_type=jnp.float32) + b1_ref[0, 0], 0.0)
        p = jnp.dot(h, w2_buf[slot], preferred_element_type=jnp.float32)

        @pl.when(j == 0)
        def _():
            acc_ref[pl.ds(row, TB), :] = p

        @pl.when(j > 0)
        def _():
            acc_ref[pl.ds(row, TB), :] += p

        @pl.when(j == NFF - 1)
        def _():
            out_ref[...] = acc_ref[pl.ds(row, TB), :] + b2_ref[0]


def _ffn(xs, W1, b1, W2, b2, meta, runinfo):
    grid_spec = pltpu.PrefetchScalarGridSpec(
        num_scalar_prefetch=2,
        grid=(NFF, NBP),
        in_specs=[
            pl.BlockSpec((NROWS, D_MODEL),
                         lambda j, b, meta, ri: (0, 0)),
            pl.BlockSpec(memory_space=pl.ANY),
            pl.BlockSpec((1, 1, 1, FFT),
                         lambda j, b, meta, ri: (meta[b, 0], j, 0, 0)),
            pl.BlockSpec(memory_space=pl.ANY),
            pl.BlockSpec((1, 1, D_MODEL),
                         lambda j, b, meta, ri: (meta[b, 0], 0, 0)),
        ],
        out_specs=pl.BlockSpec(
            (TB, D_MODEL),
            lambda j, b, meta, ri: (jnp.where(j == NFF - 1, meta[b, 1], 0),
                                    0)),
        scratch_shapes=[
            pltpu.VMEM((NROWS, D_MODEL), jnp.float32),
            pltpu.VMEM((NBUF, D_MODEL, FFT), jnp.float32),
            pltpu.VMEM((NBUF, FFT, D_MODEL), jnp.float32),
            pltpu.SemaphoreType.DMA((NBUF,)),
        ],
    )
    return pl.pallas_call(
        _ffn_body,
        grid_spec=grid_spec,
        out_shape=jax.ShapeDtypeStruct((NROWS, D_MODEL), jnp.float32),
        compiler_params=pltpu.CompilerParams(
            dimension_semantics=("arbitrary", "arbitrary"),
            vmem_limit_bytes=63_000_000),
    )(meta, runinfo, xs, W1,
      b1.reshape(E, NFF, 1, FFT), W2, b2.reshape(E, 1, D_MODEL))


def kernel(x, W1, b1, W2, b2, Wr, br):
    xf = x.reshape(S, D_MODEL)
    dest2d, meta, runinfo = _plan(xf, Wr, br.reshape(1, E))
    dest = dest2d.reshape(S)
    xs = _dispatch(xf, dest)
    outs = _ffn(xs, W1, b1, W2, b2, meta, runinfo.reshape(16))
    out = _combine(outs, dest)
    return out.reshape(B, S, D_MODEL)
